# baseline (speedup 1.0000x reference)
import functools

import jax
import jax.numpy as jnp
from jax.experimental import pallas as pl
from jax.experimental.pallas import tpu as pltpu

_ALPHA = 0.3
_CONFIDENCE = 2.0 / 3.0
_EPS = 1e-12


def _pick(sims, attrs_blk, iota):
    s = jnp.where(iota < 5, sims, -jnp.inf)
    m = jnp.max(s)
    idx = jnp.min(jnp.where(s >= m, iota, 8))
    onehot = (iota == idx).astype(jnp.float32)
    return jnp.sum(onehot * attrs_blk, axis=0, keepdims=True)


def _pool_combine_kernel(h_ref, attrs_ref, comb_ref, acc_ref, *, nb):
    i = pl.program_id(0)
    blk_sum = jnp.sum(h_ref[...], axis=0, keepdims=True)

    @pl.when(i == 0)
    def _():
        acc_ref[...] = blk_sum

    @pl.when(i > 0)
    def _():
        acc_ref[...] = acc_ref[...] + blk_sum

    @pl.when(i == nb - 1)
    def _():
        pooled = acc_ref[...]
        attrs = attrs_ref[...]
        sims = jnp.sum(pooled * attrs, axis=1, keepdims=True)
        iota = jax.lax.broadcasted_iota(jnp.int32, (8, 1), 0)
        sel = _pick(sims[0:8], attrs[0:8], iota) + _pick(sims[8:16], attrs[8:16], iota)
        comb = 0.5 * sel
        n = jnp.sqrt(jnp.sum(comb * comb))
        comb_ref[...] = comb / jnp.maximum(n, _EPS)


def _apply_kernel(h_ref, comb_ref, out_ref, *, bs, s_total):
    i = pl.program_id(0)
    h = h_ref[...]
    rn = jnp.sqrt(jnp.sum(h * h, axis=1, keepdims=True))
    row = (i * bs + jax.lax.broadcasted_iota(jnp.int32, (bs, 1), 0)).astype(
        jnp.float32
    )
    a = (_ALPHA * _CONFIDENCE) * (0.5 + 0.5 * (row / s_total))
    inv = a / jnp.maximum(rn, _EPS)
    out_ref[...] = h * (1.0 - inv) + a * comb_ref[...]


def kernel(hidden_states, attr_implication, attr_modus_ponens):
    B, S, D = hidden_states.shape
    h = hidden_states.reshape(S, D)
    attrs = (
        jnp.zeros((16, D), jnp.float32)
        .at[0:5].set(attr_implication)
        .at[8:13].set(attr_modus_ponens)
    )
    bs = 256
    nb = S // bs

    comb = pl.pallas_call(
        functools.partial(_pool_combine_kernel, nb=nb),
        grid=(nb,),
        in_specs=[
            pl.BlockSpec((bs, D), lambda i: (i, 0)),
            pl.BlockSpec((16, D), lambda i: (0, 0)),
        ],
        out_specs=pl.BlockSpec((1, D), lambda i: (0, 0)),
        out_shape=jax.ShapeDtypeStruct((1, D), jnp.float32),
        scratch_shapes=[pltpu.VMEM((1, D), jnp.float32)],
    )(h, attrs)

    out = pl.pallas_call(
        functools.partial(_apply_kernel, bs=bs, s_total=float(S)),
        grid=(nb,),
        in_specs=[
            pl.BlockSpec((bs, D), lambda i: (i, 0)),
            pl.BlockSpec((1, D), lambda i: (0, 0)),
        ],
        out_specs=pl.BlockSpec((bs, D), lambda i: (i, 0)),
        out_shape=jax.ShapeDtypeStruct((S, D), jnp.float32),
    )(h, comb)
    return out.reshape(B, S, D)


# speedup vs baseline: 1.0787x; 1.0787x over previous
import functools

import jax
import jax.numpy as jnp
from jax.experimental import pallas as pl
from jax.experimental.pallas import tpu as pltpu

_ALPHA = 0.3
_CONFIDENCE = 2.0 / 3.0
_EPS = 1e-12


def _pick(sims, attrs_blk, iota):
    s = jnp.where(iota < 5, sims, -jnp.inf)
    m = jnp.max(s)
    idx = jnp.min(jnp.where(s >= m, iota, 8))
    onehot = (iota == idx).astype(jnp.float32)
    return jnp.sum(onehot * attrs_blk, axis=0, keepdims=True)


def _fused_kernel(h_ref, attrs_ref, out_ref, acc_ref, comb_ref, *, nb, bs, s_total):
    i = pl.program_id(0)

    @pl.when(i == 0)
    def _():
        acc_ref[...] = jnp.sum(h_ref[...], axis=0, keepdims=True)

    @pl.when((i > 0) & (i < nb))
    def _():
        acc_ref[...] = acc_ref[...] + jnp.sum(h_ref[...], axis=0, keepdims=True)

    @pl.when(i == nb - 1)
    def _():
        pooled = acc_ref[...]
        attrs = attrs_ref[...]
        sims = jnp.sum(pooled * attrs, axis=1, keepdims=True)
        iota = jax.lax.broadcasted_iota(jnp.int32, (8, 1), 0)
        sel = _pick(sims[0:8], attrs[0:8], iota) + _pick(sims[8:16], attrs[8:16], iota)
        comb = 0.5 * sel
        n = jnp.sqrt(jnp.sum(comb * comb))
        comb_ref[...] = comb / jnp.maximum(n, _EPS)

    @pl.when(i >= nb)
    def _():
        j = i - nb
        h = h_ref[...]
        rn = jnp.sqrt(jnp.sum(h * h, axis=1, keepdims=True))
        row = (j * bs + jax.lax.broadcasted_iota(jnp.int32, (bs, 1), 0)).astype(
            jnp.float32
        )
        a = (_ALPHA * _CONFIDENCE) * (0.5 + 0.5 * (row / s_total))
        inv = a / jnp.maximum(rn, _EPS)
        out_ref[...] = h * (1.0 - inv) + a * comb_ref[...]


def kernel(hidden_states, attr_implication, attr_modus_ponens):
    B, S, D = hidden_states.shape
    h = hidden_states.reshape(S, D)
    attrs = (
        jnp.zeros((16, D), jnp.float32)
        .at[0:5].set(attr_implication)
        .at[8:13].set(attr_modus_ponens)
    )
    bs = 512
    nb = S // bs

    out = pl.pallas_call(
        functools.partial(_fused_kernel, nb=nb, bs=bs, s_total=float(S)),
        grid=(2 * nb,),
        in_specs=[
            pl.BlockSpec((bs, D), lambda i: (jnp.where(i < nb, i, i - nb), 0)),
            pl.BlockSpec((16, D), lambda i: (0, 0)),
        ],
        out_specs=pl.BlockSpec((bs, D), lambda i: (jnp.maximum(i - nb, 0), 0)),
        out_shape=jax.ShapeDtypeStruct((S, D), jnp.float32),
        scratch_shapes=[
            pltpu.VMEM((1, D), jnp.float32),
            pltpu.VMEM((1, D), jnp.float32),
        ],
        compiler_params=pltpu.CompilerParams(dimension_semantics=("arbitrary",)),
    )(h, attrs)
    return out.reshape(B, S, D)
